# Initial kernel scaffold; baseline (speedup 1.0000x reference)
import jax
import jax.numpy as jnp
from jax.experimental import pallas as pl


def kernel(node_feat, pos_enc, edge_feat, snorm_n, targets, edge_index, params):
    raise NotImplementedError("write your pallas kernel here")


# speedup vs baseline: 2.3050x; 2.3050x over previous
import jax
import jax.numpy as jnp
from jax import lax
from jax.experimental import pallas as pl
from jax.experimental.pallas import tpu as pltpu
from jax.experimental.pallas import tpu_sc as plsc

N = 10000
E = 320000
D = 128
POS = 8
NF = 128

NC = 2
NS = 16
NW = NC * NS
EPT = E // NW
CH = 64
NFULL = EPT // CH
REM = EPT - NFULL * CH
ZPT = 624
ZCH = 48
NZ = ZPT // ZCH
ZTAIL = N - ZPT * NS
ZTOFF = ZPT * NS

NB = 10
NBR = N // NB
EBLK = 2000
EG = E // EBLK

_F32 = jnp.float32


def _sds(shape):
    return jax.ShapeDtypeStruct(shape, _F32)


def _sc_pass_a_body(src, dst, b3e, t1, t2,
                    hat, sigpart, bnpart,
                    acc, idx_s, idx_d, idx_s2, idx_d2,
                    rows1, rows2, b3ev, hatv, sigv, bnbuf, sem1, sem2):
    c = lax.axis_index("c")
    s = lax.axis_index("s")
    wid = s * NC + c
    ebase = wid * EPT

    zero16 = jnp.zeros((16,), _F32)

    def _zrow(j, carry):
        for q in range(8):
            rows1[j, pl.ds(16 * q, 16)] = zero16
        return carry
    lax.fori_loop(0, CH, _zrow, 0)
    for k in range(NZ):
        off = pl.multiple_of(s * ZPT + ZCH * k, 8)
        pltpu.sync_copy(rows1.at[pl.ds(0, ZCH)], acc.at[pl.ds(off, ZCH)])

    @pl.when(s == NS - 1)
    def _ztail():
        pltpu.sync_copy(rows1.at[pl.ds(0, ZTAIL)], acc.at[pl.ds(ZTOFF, ZTAIL)])

    plsc.subcore_barrier()

    def _chunk(base, isr, idr, csz, bn):
        base = pl.multiple_of(base, 8)
        pltpu.sync_copy(src.at[pl.ds(base, csz)], isr)
        pltpu.sync_copy(dst.at[pl.ds(base, csz)], idr)
        g1 = pltpu.async_copy(t1.at[isr], rows1.at[pl.ds(0, csz)], sem1)
        g2 = pltpu.async_copy(t2.at[idr], rows2.at[pl.ds(0, csz)], sem2)
        pltpu.sync_copy(b3e.at[pl.ds(base, csz)], b3ev.at[pl.ds(0, csz)])
        g1.wait()
        g2.wait()

        def _row(j, bnc):
            nb = list(bnc)
            for q in range(8):
                sl = pl.ds(16 * q, 16)
                a = rows1[j, sl] + rows2[j, sl] + b3ev[j, sl]
                hatv[j, sl] = a
                sigv[j, sl] = 1.0 / (1.0 + jnp.exp(-a))
                nb[q] = nb[q] + a
                nb[8 + q] = nb[8 + q] + a * a
            return tuple(nb)

        bn = lax.fori_loop(0, csz, _row, bn)
        pltpu.sync_copy(hatv.at[pl.ds(0, csz)], hat.at[pl.ds(base, csz)])
        pltpu.sync_copy(sigv.at[pl.ds(0, csz)], acc.at[idr], add=True)
        return bn

    bn0 = tuple(jnp.zeros((16,), _F32) for _ in range(16))
    bn = lax.fori_loop(
        0, NFULL, lambda i, b: _chunk(ebase + i * CH, idx_s, idx_d, CH, b), bn0)
    bn = _chunk(ebase + NFULL * CH, idx_s2, idx_d2, REM, bn)

    for q in range(8):
        bnbuf[0, pl.ds(16 * q, 16)] = bn[q]
        bnbuf[1, pl.ds(16 * q, 16)] = bn[8 + q]
    pltpu.sync_copy(bnbuf, bnpart.at[wid])

    plsc.subcore_barrier()
    for k in range(NZ):
        off = pl.multiple_of(s * ZPT + ZCH * k, 8)
        pltpu.sync_copy(acc.at[pl.ds(off, ZCH)], hatv.at[pl.ds(0, ZCH)])
        pltpu.sync_copy(hatv.at[pl.ds(0, ZCH)], sigpart.at[c, pl.ds(off, ZCH)])

    @pl.when(s == NS - 1)
    def _dtail():
        pltpu.sync_copy(acc.at[pl.ds(ZTOFF, ZTAIL)], hatv.at[pl.ds(0, ZTAIL)])
        pltpu.sync_copy(hatv.at[pl.ds(0, ZTAIL)],
                        sigpart.at[c, pl.ds(ZTOFF, ZTAIL)])


def _sc_pass_b_body(src, dst, hat, ss, tab,
                    accpart,
                    acc, idx_s, idx_d, idx_s2, idx_d2,
                    rowt, rowss, hatv, outv, sem1, sem2):
    c = lax.axis_index("c")
    s = lax.axis_index("s")
    wid = s * NC + c
    ebase = wid * EPT

    zero16 = jnp.zeros((16,), _F32)

    def _zrow(j, carry):
        for q in range(8):
            outv[j, pl.ds(16 * q, 16)] = zero16
        return carry
    lax.fori_loop(0, CH, _zrow, 0)
    for k in range(NZ):
        off = pl.multiple_of(s * ZPT + ZCH * k, 8)
        pltpu.sync_copy(outv.at[pl.ds(0, ZCH)], acc.at[pl.ds(off, ZCH)])

    @pl.when(s == NS - 1)
    def _ztail():
        pltpu.sync_copy(outv.at[pl.ds(0, ZTAIL)], acc.at[pl.ds(ZTOFF, ZTAIL)])

    plsc.subcore_barrier()

    def _chunk(base, isr, idr, csz, carry):
        base = pl.multiple_of(base, 8)
        pltpu.sync_copy(src.at[pl.ds(base, csz)], isr)
        pltpu.sync_copy(dst.at[pl.ds(base, csz)], idr)
        g1 = pltpu.async_copy(tab.at[isr], rowt.at[pl.ds(0, csz)], sem1)
        g2 = pltpu.async_copy(ss.at[idr], rowss.at[pl.ds(0, csz)], sem2)
        pltpu.sync_copy(hat.at[pl.ds(base, csz)], hatv.at[pl.ds(0, csz)])
        g1.wait()
        g2.wait()

        def _row(j, cc):
            for q in range(8):
                sl = pl.ds(16 * q, 16)
                a = hatv[j, sl]
                sg = 1.0 / (1.0 + jnp.exp(-a))
                eta = sg / (rowss[j, sl] + 1e-6)
                outv[j, sl] = eta * rowt[j, sl]
            return cc

        lax.fori_loop(0, csz, _row, 0)
        pltpu.sync_copy(outv.at[pl.ds(0, csz)], acc.at[idr], add=True)
        return carry

    lax.fori_loop(
        0, NFULL, lambda i, cc: _chunk(ebase + i * CH, idx_s, idx_d, CH, cc), 0)
    _chunk(ebase + NFULL * CH, idx_s2, idx_d2, REM, 0)

    plsc.subcore_barrier()
    for k in range(NZ):
        off = pl.multiple_of(s * ZPT + ZCH * k, 8)
        pltpu.sync_copy(acc.at[pl.ds(off, ZCH)], hatv.at[pl.ds(0, ZCH)])
        pltpu.sync_copy(hatv.at[pl.ds(0, ZCH)], accpart.at[c, pl.ds(off, ZCH)])

    @pl.when(s == NS - 1)
    def _dtail():
        pltpu.sync_copy(acc.at[pl.ds(ZTOFF, ZTAIL)], hatv.at[pl.ds(0, ZTAIL)])
        pltpu.sync_copy(hatv.at[pl.ds(0, ZTAIL)],
                        accpart.at[c, pl.ds(ZTOFF, ZTAIL)])


_SC_MESH = plsc.VectorSubcoreMesh(
    core_axis_name="c", subcore_axis_name="s", num_cores=NC, num_subcores=NS)

_sc_pass_a = pl.kernel(
    _sc_pass_a_body,
    out_type=(_sds((E, D)), _sds((NC, N, D)), _sds((NW, 2, D))),
    mesh=_SC_MESH,
    scratch_types=[
        pltpu.VMEM_SHARED((N, D), _F32),
        pltpu.VMEM((CH,), jnp.int32), pltpu.VMEM((CH,), jnp.int32),
        pltpu.VMEM((REM,), jnp.int32), pltpu.VMEM((REM,), jnp.int32),
        pltpu.VMEM((CH, D), _F32), pltpu.VMEM((CH, D), _F32),
        pltpu.VMEM((CH, D), _F32), pltpu.VMEM((CH, D), _F32),
        pltpu.VMEM((CH, D), _F32),
        pltpu.VMEM((2, D), _F32),
        pltpu.SemaphoreType.DMA, pltpu.SemaphoreType.DMA,
    ],
)

_sc_pass_b = pl.kernel(
    _sc_pass_b_body,
    out_type=_sds((NC, N, D)),
    mesh=_SC_MESH,
    scratch_types=[
        pltpu.VMEM_SHARED((N, D), _F32),
        pltpu.VMEM((CH,), jnp.int32), pltpu.VMEM((CH,), jnp.int32),
        pltpu.VMEM((REM,), jnp.int32), pltpu.VMEM((REM,), jnp.int32),
        pltpu.VMEM((CH, D), _F32), pltpu.VMEM((CH, D), _F32),
        pltpu.VMEM((CH, D), _F32), pltpu.VMEM((CH, D), _F32),
        pltpu.SemaphoreType.DMA, pltpu.SemaphoreType.DMA,
    ],
)


def _dot(a, b):
    return jnp.dot(a, b, preferred_element_type=_F32)


def _tc_embed_body(nf, pe, ehw, ehb, epw, epb, h0, p0):
    h0[...] = _dot(nf[...], ehw[...]) + ehb[...]
    p0[...] = _dot(pe[...], epw[...]) + epb[...]


_tc_embed = pl.pallas_call(
    _tc_embed_body,
    grid=(NB,),
    in_specs=[
        pl.BlockSpec((NBR, NF), lambda i: (i, 0)),
        pl.BlockSpec((NBR, POS), lambda i: (i, 0)),
        pl.BlockSpec((NF, D), lambda i: (0, 0)),
        pl.BlockSpec((1, D), lambda i: (0, 0)),
        pl.BlockSpec((POS, D), lambda i: (0, 0)),
        pl.BlockSpec((1, D), lambda i: (0, 0)),
    ],
    out_specs=[pl.BlockSpec((NBR, D), lambda i: (i, 0))] * 2,
    out_shape=(_sds((N, D)), _sds((N, D))),
)


def _tc_tables_body(h, p, a1w, a1b, b1w, b1b, b2w, b2b, c1w, c1b,
                    a2w, a2b, c2w, c2b,
                    a1h, b1t, b2t, c1p, vt, c2p):
    hh = h[...]
    pp = p[...]
    a1 = a1w[...]
    a2 = a2w[...]
    a1h[...] = _dot(hh, a1[0:D]) + _dot(pp, a1[D:2 * D]) + a1b[...]
    b1t[...] = _dot(hh, b1w[...]) + b1b[...]
    b2t[...] = _dot(hh, b2w[...]) + b2b[...]
    c1p[...] = _dot(pp, c1w[...]) + c1b[...]
    vt[...] = _dot(hh, a2[0:D]) + _dot(pp, a2[D:2 * D]) + a2b[...]
    c2p[...] = _dot(pp, c2w[...]) + c2b[...]


_tc_tables = pl.pallas_call(
    _tc_tables_body,
    grid=(NB,),
    in_specs=[pl.BlockSpec((NBR, D), lambda i: (i, 0))] * 2 + [
        pl.BlockSpec((2 * D, D), lambda i: (0, 0)),
        pl.BlockSpec((1, D), lambda i: (0, 0)),
        pl.BlockSpec((D, D), lambda i: (0, 0)),
        pl.BlockSpec((1, D), lambda i: (0, 0)),
        pl.BlockSpec((D, D), lambda i: (0, 0)),
        pl.BlockSpec((1, D), lambda i: (0, 0)),
        pl.BlockSpec((D, D), lambda i: (0, 0)),
        pl.BlockSpec((1, D), lambda i: (0, 0)),
        pl.BlockSpec((2 * D, D), lambda i: (0, 0)),
        pl.BlockSpec((1, D), lambda i: (0, 0)),
        pl.BlockSpec((D, D), lambda i: (0, 0)),
        pl.BlockSpec((1, D), lambda i: (0, 0)),
    ],
    out_specs=[pl.BlockSpec((NBR, D), lambda i: (i, 0))] * 6,
    out_shape=tuple(_sds((N, D)) for _ in range(6)),
)


def _tc_b3e1_body(ef, ew, eb, b3w, b3b, out):
    u = _dot(ew[...], b3w[...])
    cst = _dot(eb[...], b3w[...]) + b3b[...]
    out[...] = ef[...] * u + cst


_tc_b3e1 = pl.pallas_call(
    _tc_b3e1_body,
    grid=(EG,),
    in_specs=[
        pl.BlockSpec((EBLK, 1), lambda i: (i, 0)),
        pl.BlockSpec((1, D), lambda i: (0, 0)),
        pl.BlockSpec((1, D), lambda i: (0, 0)),
        pl.BlockSpec((D, D), lambda i: (0, 0)),
        pl.BlockSpec((1, D), lambda i: (0, 0)),
    ],
    out_specs=pl.BlockSpec((EBLK, D), lambda i: (i, 0)),
    out_shape=_sds((E, D)),
)


def _tc_combine_body(sp, bp, ss, mv):
    spv = sp[...]
    ss[...] = spv[0] + spv[1]
    bpv = bp[...]
    m = jnp.sum(bpv[:, 0, :], axis=0) / float(E)
    q = jnp.sum(bpv[:, 1, :], axis=0) / float(E)
    mv[...] = jnp.stack([m, q - m * m], axis=0)


_tc_combine = pl.pallas_call(
    _tc_combine_body,
    out_shape=(_sds((N, D)), _sds((2, D))),
)


def _tc_b3e2_body(hat, ef, mv, ew, eb, ge, be, b3w, b3b, out):
    mvv = mv[...]
    m = mvv[0:1, :]
    v = mvv[1:2, :]
    xn = (hat[...] - m) / jnp.sqrt(v + 1e-5) * ge[...] + be[...]
    e2 = ef[...] * ew[...] + eb[...] + jnp.maximum(xn, 0.0)
    out[...] = _dot(e2, b3w[...]) + b3b[...]


_tc_b3e2 = pl.pallas_call(
    _tc_b3e2_body,
    grid=(EG,),
    in_specs=[
        pl.BlockSpec((EBLK, D), lambda i: (i, 0)),
        pl.BlockSpec((EBLK, 1), lambda i: (i, 0)),
        pl.BlockSpec((2, D), lambda i: (0, 0)),
        pl.BlockSpec((1, D), lambda i: (0, 0)),
        pl.BlockSpec((1, D), lambda i: (0, 0)),
        pl.BlockSpec((1, D), lambda i: (0, 0)),
        pl.BlockSpec((1, D), lambda i: (0, 0)),
        pl.BlockSpec((D, D), lambda i: (0, 0)),
        pl.BlockSpec((1, D), lambda i: (0, 0)),
    ],
    out_specs=pl.BlockSpec((EBLK, D), lambda i: (i, 0)),
    out_shape=_sds((E, D)),
)


def _tc_hupd_body(a1h, hacc, sn, hin, g, b, out):
    ha = hacc[...]
    t = (a1h[...] + ha[0] + ha[1]) * sn[...]
    m = jnp.mean(t, axis=0, keepdims=True)
    v = jnp.mean(t * t, axis=0, keepdims=True) - m * m
    out[...] = hin[...] + jnp.maximum(
        (t - m) / jnp.sqrt(v + 1e-5) * g[...] + b[...], 0.0)


_tc_hupd = pl.pallas_call(_tc_hupd_body, out_shape=_sds((N, D)))


def _tc_pupd_body(c1p, pacc, pin, out):
    pa = pacc[...]
    out[...] = pin[...] + jnp.tanh(c1p[...] + pa[0] + pa[1])


_tc_pupd = pl.pallas_call(_tc_pupd_body, out_shape=_sds((N, D)))


def _tc_head_body(a1h2, hacc, sn, h1, g, b, c1p2, pacc, p1,
                  pow_, pob, whpw, whpb, w1, b1_, w2, b2_, w3, b3_, out):
    ha = hacc[...]
    t = (a1h2[...] + ha[0] + ha[1]) * sn[...]
    m = jnp.mean(t, axis=0, keepdims=True)
    v = jnp.mean(t * t, axis=0, keepdims=True) - m * m
    h3 = h1[...] + jnp.maximum(
        (t - m) / jnp.sqrt(v + 1e-5) * g[...] + b[...], 0.0)
    pa = pacc[...]
    p3 = p1[...] + jnp.tanh(c1p2[...] + pa[0] + pa[1])
    pp = _dot(p3, pow_[...]) + pob[...]
    pp = pp - jnp.mean(pp, axis=0, keepdims=True)
    pp = pp / jnp.sqrt(jnp.sum(pp * pp, axis=0, keepdims=True))
    whp = whpw[...]
    hp0 = _dot(h3[0:1], whp[0:D]) + _dot(pp[0:1], whp[D:D + POS]) + whpb[...]
    y = jnp.maximum(_dot(hp0, w1[...]) + b1_[...], 0.0)
    y = jnp.maximum(_dot(y, w2[...]) + b2_[...], 0.0)
    y = _dot(y, w3[...]) + b3_[...]
    out[...] = jnp.broadcast_to(y, (8, 128))


_tc_head = pl.pallas_call(_tc_head_body, out_shape=_sds((8, 128)))


def kernel(node_feat, pos_enc, edge_feat, snorm_n, targets, edge_index, params):
    lp1, lp2 = params['layers'][0], params['layers'][1]
    src = edge_index[0]
    dst = edge_index[1]

    def r2(x):
        return x.reshape(1, -1)

    h0, p0 = _tc_embed(node_feat, pos_enc,
                       params['emb_h_W'], r2(params['emb_h_b']),
                       params['emb_p_W'], r2(params['emb_p_b']))

    def tables(h, p, lp):
        return _tc_tables(h, p,
                          lp['A1_W'], r2(lp['A1_b']),
                          lp['B1_W'], r2(lp['B1_b']),
                          lp['B2_W'], r2(lp['B2_b']),
                          lp['C1_W'], r2(lp['C1_b']),
                          lp['A2_W'], r2(lp['A2_b']),
                          lp['C2_W'], r2(lp['C2_b']))

    a1h1, b11, b21, c1p1, v1, c2p1 = tables(h0, p0, lp1)
    b3e1 = _tc_b3e1(edge_feat, params['emb_e_W'], r2(params['emb_e_b']),
                    lp1['B3_W'], r2(lp1['B3_b']))

    hat1, sigp1, bnp1 = _sc_pass_a(src, dst, b3e1, b11, b21)
    ss1, mv1 = _tc_combine(sigp1, bnp1)
    hacc1 = _sc_pass_b(src, dst, hat1, ss1, v1)
    pacc1 = _sc_pass_b(src, dst, hat1, ss1, c2p1)

    h1 = _tc_hupd(a1h1, hacc1, snorm_n, h0, r2(lp1['bn_h_g']), r2(lp1['bn_h_b']))
    p1 = _tc_pupd(c1p1, pacc1, p0)
    b3e2 = _tc_b3e2(hat1, edge_feat, mv1,
                    params['emb_e_W'], r2(params['emb_e_b']),
                    r2(lp1['bn_e_g']), r2(lp1['bn_e_b']),
                    lp2['B3_W'], r2(lp2['B3_b']))

    a1h2, b12, b22, c1p2, v2, c2p2 = tables(h1, p1, lp2)
    hat2, sigp2, bnp2 = _sc_pass_a(src, dst, b3e2, b12, b22)
    ss2, _unused = _tc_combine(sigp2, bnp2)
    hacc2 = _sc_pass_b(src, dst, hat2, ss2, v2)
    pacc2 = _sc_pass_b(src, dst, hat2, ss2, c2p2)

    out = _tc_head(a1h2, hacc2, snorm_n, h1,
                   r2(lp2['bn_h_g']), r2(lp2['bn_h_b']),
                   c1p2, pacc2, p1,
                   params['p_out_W'], r2(params['p_out_b']),
                   params['Whp_W'], r2(params['Whp_b']),
                   params['mlp'][0][0], r2(params['mlp'][0][1]),
                   params['mlp'][1][0], r2(params['mlp'][1][1]),
                   params['mlp'][2][0], r2(params['mlp'][2][1]))
    scores = out[0:1, 0:1]
    return (scores, targets)


# speedup vs baseline: 2.6645x; 1.1560x over previous
import functools

import jax
import jax.numpy as jnp
from jax import lax
from jax.experimental import pallas as pl
from jax.experimental.pallas import tpu as pltpu
from jax.experimental.pallas import tpu_sc as plsc

N = 10000
E = 320000
D = 128
POS = 8
NF = 128

NC = 2
NS = 16
NW = NC * NS
EPT = E // NW

CHA = 64
NFA = EPT // CHA
REM = 16
CHB = 128
NFB = (EPT - REM) // CHB

ZPT = 624
ZCH = 48
NZ = ZPT // ZCH
ZTAIL = N - ZPT * NS
ZTOFF = ZPT * NS

NB = 10
NBR = N // NB
EBLK = 2000
EG = E // EBLK

_F32 = jnp.float32


def _sds(shape):
    return jax.ShapeDtypeStruct(shape, _F32)


def _zero_acc(zbuf, acc, s):
    zero16 = jnp.zeros((16,), _F32)

    def _zrow(j, carry):
        for q in range(8):
            zbuf[j, pl.ds(16 * q, 16)] = zero16
        return carry
    lax.fori_loop(0, ZCH, _zrow, 0)
    for k in range(NZ):
        off = pl.multiple_of(s * ZPT + ZCH * k, 8)
        pltpu.sync_copy(zbuf.at[pl.ds(0, ZCH)], acc.at[pl.ds(off, ZCH)])

    @pl.when(s == NS - 1)
    def _ztail():
        pltpu.sync_copy(zbuf.at[pl.ds(0, ZTAIL)], acc.at[pl.ds(ZTOFF, ZTAIL)])


def _dump_acc(bounce, acc, out, c, s):
    for k in range(NZ):
        off = pl.multiple_of(s * ZPT + ZCH * k, 8)
        pltpu.sync_copy(acc.at[pl.ds(off, ZCH)], bounce.at[pl.ds(0, ZCH)])
        pltpu.sync_copy(bounce.at[pl.ds(0, ZCH)], out.at[c, pl.ds(off, ZCH)])

    @pl.when(s == NS - 1)
    def _dtail():
        pltpu.sync_copy(acc.at[pl.ds(ZTOFF, ZTAIL)], bounce.at[pl.ds(0, ZTAIL)])
        pltpu.sync_copy(bounce.at[pl.ds(0, ZTAIL)], out.at[c, pl.ds(ZTOFF, ZTAIL)])


def _sc_pass_a_body(write_hat, src, dst, b3e, t1, t2, *args):
    if write_hat:
        (hat, sig, sigpart, bnpart,
         acc, isA, idA, isB, idB, is2, id2,
         r1A, r1B, r2A, r2B, bv, bnbuf, s1A, s2A, s1B, s2B) = args
    else:
        (sig, sigpart, bnpart,
         acc, isA, idA, isB, idB, is2, id2,
         r1A, r1B, r2A, r2B, bv, bnbuf, s1A, s2A, s1B, s2B) = args
        hat = None
    c = lax.axis_index("c")
    s = lax.axis_index("s")
    wid = s * NC + c
    ebase = wid * EPT

    _zero_acc(r1A, acc, s)
    plsc.subcore_barrier()

    def _issue(base, isr, idr, g1, g2, sm1, sm2):
        base = pl.multiple_of(base, 8)
        pltpu.sync_copy(src.at[pl.ds(base, CHA)], isr)
        pltpu.sync_copy(dst.at[pl.ds(base, CHA)], idr)
        pltpu.async_copy(t1.at[isr], g1, sm1)
        pltpu.async_copy(t2.at[idr], g2, sm2)

    def _wait(isr, idr, g1, g2, sm1, sm2):
        pltpu.make_async_copy(t1.at[isr], g1, sm1).wait()
        pltpu.make_async_copy(t2.at[idr], g2, sm2).wait()

    def _process(base, isr, idr, g1, g2, sm1, sm2, bn):
        base = pl.multiple_of(base, 8)
        pltpu.sync_copy(b3e.at[pl.ds(base, CHA)], bv)
        _wait(isr, idr, g1, g2, sm1, sm2)

        def _row1(j, bnc):
            nb = list(bnc)
            for q in range(8):
                sl = pl.ds(16 * q, 16)
                a = g1[j, sl] + g2[j, sl] + bv[j, sl]
                bv[j, sl] = a
                nb[q] = nb[q] + a
                nb[8 + q] = nb[8 + q] + a * a
            return tuple(nb)
        bn = lax.fori_loop(0, CHA, _row1, bn)
        if hat is not None:
            pltpu.sync_copy(bv, hat.at[pl.ds(base, CHA)])

        def _row2(j, carry):
            for q in range(8):
                sl = pl.ds(16 * q, 16)
                bv[j, sl] = 1.0 / (1.0 + jnp.exp(-bv[j, sl]))
            return carry
        lax.fori_loop(0, CHA, _row2, 0)
        pltpu.sync_copy(bv, sig.at[pl.ds(base, CHA)])
        pltpu.sync_copy(bv, acc.at[idr], add=True)
        return bn

    bn0 = tuple(jnp.zeros((16,), _F32) for _ in range(16))
    _issue(ebase, isA, idA, r1A, r2A, s1A, s2A)

    def _pair(k, bn):
        i0 = ebase + (2 * k) * CHA
        i1 = i0 + CHA
        inext = jnp.minimum(i1 + CHA, ebase + (NFA - 2) * CHA)
        _issue(i1, isB, idB, r1B, r2B, s1B, s2B)
        bn = _process(i0, isA, idA, r1A, r2A, s1A, s2A, bn)
        _issue(inext, isA, idA, r1A, r2A, s1A, s2A)
        bn = _process(i1, isB, idB, r1B, r2B, s1B, s2B, bn)
        return bn

    bn = lax.fori_loop(0, NFA // 2, _pair, bn0)
    _wait(isA, idA, r1A, r2A, s1A, s2A)

    rbase = pl.multiple_of(ebase + NFA * CHA, 8)
    pltpu.sync_copy(src.at[pl.ds(rbase, REM)], is2)
    pltpu.sync_copy(dst.at[pl.ds(rbase, REM)], id2)
    pltpu.async_copy(t1.at[is2], r1A.at[pl.ds(0, REM)], s1A)
    pltpu.async_copy(t2.at[id2], r2A.at[pl.ds(0, REM)], s2A)
    pltpu.sync_copy(b3e.at[pl.ds(rbase, REM)], bv.at[pl.ds(0, REM)])
    pltpu.make_async_copy(t1.at[is2], r1A.at[pl.ds(0, REM)], s1A).wait()
    pltpu.make_async_copy(t2.at[id2], r2A.at[pl.ds(0, REM)], s2A).wait()

    def _rrow1(j, bnc):
        nb = list(bnc)
        for q in range(8):
            sl = pl.ds(16 * q, 16)
            a = r1A[j, sl] + r2A[j, sl] + bv[j, sl]
            bv[j, sl] = a
            nb[q] = nb[q] + a
            nb[8 + q] = nb[8 + q] + a * a
        return tuple(nb)
    bn = lax.fori_loop(0, REM, _rrow1, bn)
    if hat is not None:
        pltpu.sync_copy(bv.at[pl.ds(0, REM)], hat.at[pl.ds(rbase, REM)])

    def _rrow2(j, carry):
        for q in range(8):
            sl = pl.ds(16 * q, 16)
            bv[j, sl] = 1.0 / (1.0 + jnp.exp(-bv[j, sl]))
        return carry
    lax.fori_loop(0, REM, _rrow2, 0)
    pltpu.sync_copy(bv.at[pl.ds(0, REM)], sig.at[pl.ds(rbase, REM)])
    pltpu.sync_copy(bv.at[pl.ds(0, REM)], acc.at[id2], add=True)

    for q in range(8):
        bnbuf[0, pl.ds(16 * q, 16)] = bn[q]
        bnbuf[1, pl.ds(16 * q, 16)] = bn[8 + q]
    pltpu.sync_copy(bnbuf, bnpart.at[wid])

    plsc.subcore_barrier()
    _dump_acc(r1A, acc, sigpart, c, s)


def _sc_pass_b_body(src, dst, sig, tab,
                    accpart,
                    acc, isA, idA, isB, idB, is2, id2,
                    rtA, rtB, sgv, s1A, s1B):
    c = lax.axis_index("c")
    s = lax.axis_index("s")
    ebase = (s * NC + c) * EPT

    _zero_acc(sgv, acc, s)
    plsc.subcore_barrier()

    def _issue(base, isr, idr, gt, sm):
        base = pl.multiple_of(base, 8)
        pltpu.sync_copy(src.at[pl.ds(base, CHB)], isr)
        pltpu.sync_copy(dst.at[pl.ds(base, CHB)], idr)
        pltpu.async_copy(tab.at[isr], gt, sm)

    def _process(base, isr, idr, gt, sm):
        base = pl.multiple_of(base, 8)
        pltpu.sync_copy(sig.at[pl.ds(base, CHB)], sgv)
        pltpu.make_async_copy(tab.at[isr], gt, sm).wait()

        def _row(j, carry):
            for q in range(8):
                sl = pl.ds(16 * q, 16)
                sgv[j, sl] = sgv[j, sl] * gt[j, sl]
            return carry
        lax.fori_loop(0, CHB, _row, 0)
        pltpu.sync_copy(sgv, acc.at[idr], add=True)

    _issue(ebase, isA, idA, rtA, s1A)

    def _pair(k, carry):
        i0 = ebase + (2 * k) * CHB
        i1 = i0 + CHB
        inext = jnp.minimum(i1 + CHB, ebase + (NFB - 2) * CHB)
        _issue(i1, isB, idB, rtB, s1B)
        _process(i0, isA, idA, rtA, s1A)
        _issue(inext, isA, idA, rtA, s1A)
        _process(i1, isB, idB, rtB, s1B)
        return carry

    lax.fori_loop(0, NFB // 2, _pair, 0)
    pltpu.make_async_copy(tab.at[isA], rtA, s1A).wait()

    rbase = pl.multiple_of(ebase + NFB * CHB, 8)
    pltpu.sync_copy(src.at[pl.ds(rbase, REM)], is2)
    pltpu.sync_copy(dst.at[pl.ds(rbase, REM)], id2)
    pltpu.async_copy(tab.at[is2], rtA.at[pl.ds(0, REM)], s1A)
    pltpu.sync_copy(sig.at[pl.ds(rbase, REM)], sgv.at[pl.ds(0, REM)])
    pltpu.make_async_copy(tab.at[is2], rtA.at[pl.ds(0, REM)], s1A).wait()

    def _rrow(j, carry):
        for q in range(8):
            sl = pl.ds(16 * q, 16)
            sgv[j, sl] = sgv[j, sl] * rtA[j, sl]
        return carry
    lax.fori_loop(0, REM, _rrow, 0)
    pltpu.sync_copy(sgv.at[pl.ds(0, REM)], acc.at[id2], add=True)

    plsc.subcore_barrier()
    _dump_acc(sgv, acc, accpart, c, s)


_SC_MESH = plsc.VectorSubcoreMesh(
    core_axis_name="c", subcore_axis_name="s", num_cores=NC, num_subcores=NS)

_A_SCRATCH = [
    pltpu.VMEM_SHARED((N, D), _F32),
    pltpu.VMEM((CHA,), jnp.int32), pltpu.VMEM((CHA,), jnp.int32),
    pltpu.VMEM((CHA,), jnp.int32), pltpu.VMEM((CHA,), jnp.int32),
    pltpu.VMEM((REM,), jnp.int32), pltpu.VMEM((REM,), jnp.int32),
    pltpu.VMEM((CHA, D), _F32), pltpu.VMEM((CHA, D), _F32),
    pltpu.VMEM((CHA, D), _F32), pltpu.VMEM((CHA, D), _F32),
    pltpu.VMEM((CHA, D), _F32),
    pltpu.VMEM((2, D), _F32),
    pltpu.SemaphoreType.DMA, pltpu.SemaphoreType.DMA,
    pltpu.SemaphoreType.DMA, pltpu.SemaphoreType.DMA,
]

_sc_pass_a1 = pl.kernel(
    functools.partial(_sc_pass_a_body, True),
    out_type=(_sds((E, D)), _sds((E, D)), _sds((NC, N, D)), _sds((NW, 2, D))),
    mesh=_SC_MESH,
    scratch_types=list(_A_SCRATCH),
)

_sc_pass_a2 = pl.kernel(
    functools.partial(_sc_pass_a_body, False),
    out_type=(_sds((E, D)), _sds((NC, N, D)), _sds((NW, 2, D))),
    mesh=_SC_MESH,
    scratch_types=list(_A_SCRATCH),
)

_sc_pass_b = pl.kernel(
    _sc_pass_b_body,
    out_type=_sds((NC, N, D)),
    mesh=_SC_MESH,
    scratch_types=[
        pltpu.VMEM_SHARED((N, D), _F32),
        pltpu.VMEM((CHB,), jnp.int32), pltpu.VMEM((CHB,), jnp.int32),
        pltpu.VMEM((CHB,), jnp.int32), pltpu.VMEM((CHB,), jnp.int32),
        pltpu.VMEM((REM,), jnp.int32), pltpu.VMEM((REM,), jnp.int32),
        pltpu.VMEM((CHB, D), _F32), pltpu.VMEM((CHB, D), _F32),
        pltpu.VMEM((CHB, D), _F32),
        pltpu.SemaphoreType.DMA, pltpu.SemaphoreType.DMA,
    ],
)


def _dot(a, b):
    return jnp.dot(a, b, preferred_element_type=_F32)


def _tc_embed_body(nf, pe, ehw, ehb, epw, epb, h0, p0):
    h0[...] = _dot(nf[...], ehw[...]) + ehb[...]
    p0[...] = _dot(pe[...], epw[...]) + epb[...]


_tc_embed = pl.pallas_call(
    _tc_embed_body,
    grid=(NB,),
    in_specs=[
        pl.BlockSpec((NBR, NF), lambda i: (i, 0)),
        pl.BlockSpec((NBR, POS), lambda i: (i, 0)),
        pl.BlockSpec((NF, D), lambda i: (0, 0)),
        pl.BlockSpec((1, D), lambda i: (0, 0)),
        pl.BlockSpec((POS, D), lambda i: (0, 0)),
        pl.BlockSpec((1, D), lambda i: (0, 0)),
    ],
    out_specs=[pl.BlockSpec((NBR, D), lambda i: (i, 0))] * 2,
    out_shape=(_sds((N, D)), _sds((N, D))),
)


def _tc_tables_body(h, p, a1w, a1b, b1w, b1b, b2w, b2b, c1w, c1b,
                    a2w, a2b, c2w, c2b,
                    a1h, b1t, b2t, c1p, vt, c2p):
    hh = h[...]
    pp = p[...]
    a1 = a1w[...]
    a2 = a2w[...]
    a1h[...] = _dot(hh, a1[0:D]) + _dot(pp, a1[D:2 * D]) + a1b[...]
    b1t[...] = _dot(hh, b1w[...]) + b1b[...]
    b2t[...] = _dot(hh, b2w[...]) + b2b[...]
    c1p[...] = _dot(pp, c1w[...]) + c1b[...]
    vt[...] = _dot(hh, a2[0:D]) + _dot(pp, a2[D:2 * D]) + a2b[...]
    c2p[...] = _dot(pp, c2w[...]) + c2b[...]


_tc_tables = pl.pallas_call(
    _tc_tables_body,
    grid=(NB,),
    in_specs=[pl.BlockSpec((NBR, D), lambda i: (i, 0))] * 2 + [
        pl.BlockSpec((2 * D, D), lambda i: (0, 0)),
        pl.BlockSpec((1, D), lambda i: (0, 0)),
        pl.BlockSpec((D, D), lambda i: (0, 0)),
        pl.BlockSpec((1, D), lambda i: (0, 0)),
        pl.BlockSpec((D, D), lambda i: (0, 0)),
        pl.BlockSpec((1, D), lambda i: (0, 0)),
        pl.BlockSpec((D, D), lambda i: (0, 0)),
        pl.BlockSpec((1, D), lambda i: (0, 0)),
        pl.BlockSpec((2 * D, D), lambda i: (0, 0)),
        pl.BlockSpec((1, D), lambda i: (0, 0)),
        pl.BlockSpec((D, D), lambda i: (0, 0)),
        pl.BlockSpec((1, D), lambda i: (0, 0)),
    ],
    out_specs=[pl.BlockSpec((NBR, D), lambda i: (i, 0))] * 6,
    out_shape=tuple(_sds((N, D)) for _ in range(6)),
)


def _tc_b3e1_body(ef, ew, eb, b3w, b3b, out):
    u = _dot(ew[...], b3w[...])
    cst = _dot(eb[...], b3w[...]) + b3b[...]
    out[...] = ef[...] * u + cst


_tc_b3e1 = pl.pallas_call(
    _tc_b3e1_body,
    grid=(EG,),
    in_specs=[
        pl.BlockSpec((EBLK, 1), lambda i: (i, 0)),
        pl.BlockSpec((1, D), lambda i: (0, 0)),
        pl.BlockSpec((1, D), lambda i: (0, 0)),
        pl.BlockSpec((D, D), lambda i: (0, 0)),
        pl.BlockSpec((1, D), lambda i: (0, 0)),
    ],
    out_specs=pl.BlockSpec((EBLK, D), lambda i: (i, 0)),
    out_shape=_sds((E, D)),
)


def _tc_combine_body(sp, bp, ss, mv):
    spv = sp[...]
    ss[...] = spv[0] + spv[1]
    bpv = bp[...]
    m = jnp.sum(bpv[:, 0, :], axis=0) / float(E)
    q = jnp.sum(bpv[:, 1, :], axis=0) / float(E)
    mv[...] = jnp.stack([m, q - m * m], axis=0)


_tc_combine = pl.pallas_call(
    _tc_combine_body,
    out_shape=(_sds((N, D)), _sds((2, D))),
)


def _tc_b3e2_body(hat, ef, mv, ew, eb, ge, be, b3w, b3b, out):
    mvv = mv[...]
    m = mvv[0:1, :]
    v = mvv[1:2, :]
    xn = (hat[...] - m) / jnp.sqrt(v + 1e-5) * ge[...] + be[...]
    e2 = ef[...] * ew[...] + eb[...] + jnp.maximum(xn, 0.0)
    out[...] = _dot(e2, b3w[...]) + b3b[...]


_tc_b3e2 = pl.pallas_call(
    _tc_b3e2_body,
    grid=(EG,),
    in_specs=[
        pl.BlockSpec((EBLK, D), lambda i: (i, 0)),
        pl.BlockSpec((EBLK, 1), lambda i: (i, 0)),
        pl.BlockSpec((2, D), lambda i: (0, 0)),
        pl.BlockSpec((1, D), lambda i: (0, 0)),
        pl.BlockSpec((1, D), lambda i: (0, 0)),
        pl.BlockSpec((1, D), lambda i: (0, 0)),
        pl.BlockSpec((1, D), lambda i: (0, 0)),
        pl.BlockSpec((D, D), lambda i: (0, 0)),
        pl.BlockSpec((1, D), lambda i: (0, 0)),
    ],
    out_specs=pl.BlockSpec((EBLK, D), lambda i: (i, 0)),
    out_shape=_sds((E, D)),
)


def _tc_hupd_body(a1h, hacc, ss, sn, hin, g, b, out):
    ha = hacc[...]
    t = (a1h[...] + (ha[0] + ha[1]) / (ss[...] + 1e-6)) * sn[...]
    m = jnp.mean(t, axis=0, keepdims=True)
    v = jnp.mean(t * t, axis=0, keepdims=True) - m * m
    out[...] = hin[...] + jnp.maximum(
        (t - m) / jnp.sqrt(v + 1e-5) * g[...] + b[...], 0.0)


_tc_hupd = pl.pallas_call(_tc_hupd_body, out_shape=_sds((N, D)))


def _tc_pupd_body(c1p, pacc, ss, pin, out):
    pa = pacc[...]
    out[...] = pin[...] + jnp.tanh(
        c1p[...] + (pa[0] + pa[1]) / (ss[...] + 1e-6))


_tc_pupd = pl.pallas_call(_tc_pupd_body, out_shape=_sds((N, D)))


def _tc_head_body(a1h2, hacc, ss2, sn, h1, g, b, p3r,
                  pow_, pob, whpw, whpb, w1, b1_, w2, b2_, w3, b3_, out):
    ha = hacc[...]
    ssv = ss2[...] + 1e-6
    t = (a1h2[...] + (ha[0] + ha[1]) / ssv) * sn[...]
    m = jnp.mean(t, axis=0, keepdims=True)
    v = jnp.mean(t * t, axis=0, keepdims=True) - m * m
    h3 = h1[...] + jnp.maximum(
        (t - m) / jnp.sqrt(v + 1e-5) * g[...] + b[...], 0.0)
    p3 = p3r[...]
    pp = _dot(p3, pow_[...]) + pob[...]
    pp = pp - jnp.mean(pp, axis=0, keepdims=True)
    pp = pp / jnp.sqrt(jnp.sum(pp * pp, axis=0, keepdims=True))
    whp = whpw[...]
    hp0 = _dot(h3[0:1], whp[0:D]) + _dot(pp[0:1], whp[D:D + POS]) + whpb[...]
    y = jnp.maximum(_dot(hp0, w1[...]) + b1_[...], 0.0)
    y = jnp.maximum(_dot(y, w2[...]) + b2_[...], 0.0)
    y = _dot(y, w3[...]) + b3_[...]
    out[...] = jnp.broadcast_to(y, (8, 128))


_tc_head = pl.pallas_call(_tc_head_body, out_shape=_sds((8, 128)))


def kernel(node_feat, pos_enc, edge_feat, snorm_n, targets, edge_index, params):
    lp1, lp2 = params['layers'][0], params['layers'][1]
    src = edge_index[0]
    dst = edge_index[1]

    def r2(x):
        return x.reshape(1, -1)

    h0, p0 = _tc_embed(node_feat, pos_enc,
                       params['emb_h_W'], r2(params['emb_h_b']),
                       params['emb_p_W'], r2(params['emb_p_b']))

    def tables(h, p, lp):
        return _tc_tables(h, p,
                          lp['A1_W'], r2(lp['A1_b']),
                          lp['B1_W'], r2(lp['B1_b']),
                          lp['B2_W'], r2(lp['B2_b']),
                          lp['C1_W'], r2(lp['C1_b']),
                          lp['A2_W'], r2(lp['A2_b']),
                          lp['C2_W'], r2(lp['C2_b']))

    a1h1, b11, b21, c1p1, v1, c2p1 = tables(h0, p0, lp1)
    b3e1 = _tc_b3e1(edge_feat, params['emb_e_W'], r2(params['emb_e_b']),
                    lp1['B3_W'], r2(lp1['B3_b']))

    hat1, sig1, sigp1, bnp1 = _sc_pass_a1(src, dst, b3e1, b11, b21)
    ss1, mv1 = _tc_combine(sigp1, bnp1)
    hacc1 = _sc_pass_b(src, dst, sig1, v1)
    pacc1 = _sc_pass_b(src, dst, sig1, c2p1)

    h1 = _tc_hupd(a1h1, hacc1, ss1, snorm_n, h0,
                  r2(lp1['bn_h_g']), r2(lp1['bn_h_b']))
    p1 = _tc_pupd(c1p1, pacc1, ss1, p0)
    b3e2 = _tc_b3e2(hat1, edge_feat, mv1,
                    params['emb_e_W'], r2(params['emb_e_b']),
                    r2(lp1['bn_e_g']), r2(lp1['bn_e_b']),
                    lp2['B3_W'], r2(lp2['B3_b']))

    a1h2, b12, b22, c1p2, v2, c2p2 = tables(h1, p1, lp2)
    sig2, sigp2, bnp2 = _sc_pass_a2(src, dst, b3e2, b12, b22)
    ss2, _unused = _tc_combine(sigp2, bnp2)
    hacc2 = _sc_pass_b(src, dst, sig2, v2)
    pacc2 = _sc_pass_b(src, dst, sig2, c2p2)

    p3 = _tc_pupd(c1p2, pacc2, ss2, p1)
    out = _tc_head(a1h2, hacc2, ss2, snorm_n, h1,
                   r2(lp2['bn_h_g']), r2(lp2['bn_h_b']),
                   p3,
                   params['p_out_W'], r2(params['p_out_b']),
                   params['Whp_W'], r2(params['Whp_b']),
                   params['mlp'][0][0], r2(params['mlp'][0][1]),
                   params['mlp'][1][0], r2(params['mlp'][1][1]),
                   params['mlp'][2][0], r2(params['mlp'][2][1]))
    scores = out[0:1, 0:1]
    return (scores, targets)


# speedup vs baseline: 2.6655x; 1.0004x over previous
import functools

import jax
import jax.numpy as jnp
from jax import lax
from jax.experimental import pallas as pl
from jax.experimental.pallas import tpu as pltpu
from jax.experimental.pallas import tpu_sc as plsc

N = 10000
E = 320000
D = 128
POS = 8
NF = 128

NC = 2
NS = 16
NW = NC * NS
EPT = E // NW

CHA = 64
NFA = EPT // CHA
REM = 16
CHB = 128
NFB = (EPT - REM) // CHB

ZPT = 624
ZCH = 48
NZ = ZPT // ZCH
ZTAIL = N - ZPT * NS
ZTOFF = ZPT * NS

NB = 10
NBR = N // NB
EBLK = 2000
EG = E // EBLK

_F32 = jnp.float32


def _sds(shape):
    return jax.ShapeDtypeStruct(shape, _F32)


def _zero_acc(zbuf, acc, s):
    zero16 = jnp.zeros((16,), _F32)

    def _zrow(j, carry):
        for q in range(8):
            zbuf[j, pl.ds(16 * q, 16)] = zero16
        return carry
    lax.fori_loop(0, ZCH, _zrow, 0)
    for k in range(NZ):
        off = pl.multiple_of(s * ZPT + ZCH * k, 8)
        pltpu.sync_copy(zbuf.at[pl.ds(0, ZCH)], acc.at[pl.ds(off, ZCH)])

    @pl.when(s == NS - 1)
    def _ztail():
        pltpu.sync_copy(zbuf.at[pl.ds(0, ZTAIL)], acc.at[pl.ds(ZTOFF, ZTAIL)])


def _dump_acc(bounce, acc, out, c, s):
    for k in range(NZ):
        off = pl.multiple_of(s * ZPT + ZCH * k, 8)
        pltpu.sync_copy(acc.at[pl.ds(off, ZCH)], bounce.at[pl.ds(0, ZCH)])
        pltpu.sync_copy(bounce.at[pl.ds(0, ZCH)], out.at[c, pl.ds(off, ZCH)])

    @pl.when(s == NS - 1)
    def _dtail():
        pltpu.sync_copy(acc.at[pl.ds(ZTOFF, ZTAIL)], bounce.at[pl.ds(0, ZTAIL)])
        pltpu.sync_copy(bounce.at[pl.ds(0, ZTAIL)], out.at[c, pl.ds(ZTOFF, ZTAIL)])


def _sc_pass_a_body(write_hat, src, dst, b3e, t1, t2, *args):
    if write_hat:
        (hat, sig, sigpart, bnpart,
         acc, isA, idA, isB, idB, is2, id2,
         r1A, r1B, r2A, r2B, bv, bnbuf, s1A, s2A, s1B, s2B) = args
    else:
        (sig, sigpart, bnpart,
         acc, isA, idA, isB, idB, is2, id2,
         r1A, r1B, r2A, r2B, bv, bnbuf, s1A, s2A, s1B, s2B) = args
        hat = None
    c = lax.axis_index("c")
    s = lax.axis_index("s")
    wid = s * NC + c
    ebase = wid * EPT

    _zero_acc(r1A, acc, s)
    plsc.subcore_barrier()

    def _issue(base, isr, idr, g1, g2, sm1, sm2):
        base = pl.multiple_of(base, 8)
        pltpu.sync_copy(src.at[pl.ds(base, CHA)], isr)
        pltpu.sync_copy(dst.at[pl.ds(base, CHA)], idr)
        pltpu.async_copy(t1.at[isr], g1, sm1)
        pltpu.async_copy(t2.at[idr], g2, sm2)

    def _wait(isr, idr, g1, g2, sm1, sm2):
        pltpu.make_async_copy(t1.at[isr], g1, sm1).wait()
        pltpu.make_async_copy(t2.at[idr], g2, sm2).wait()

    def _process(base, isr, idr, g1, g2, sm1, sm2, bn):
        base = pl.multiple_of(base, 8)
        pltpu.sync_copy(b3e.at[pl.ds(base, CHA)], bv)
        _wait(isr, idr, g1, g2, sm1, sm2)

        def _row1(j, bnc):
            nb = list(bnc)
            for q in range(8):
                sl = pl.ds(16 * q, 16)
                a = g1[j, sl] + g2[j, sl] + bv[j, sl]
                bv[j, sl] = a
                nb[q] = nb[q] + a
                nb[8 + q] = nb[8 + q] + a * a
            return tuple(nb)
        bn = lax.fori_loop(0, CHA, _row1, bn)
        if hat is not None:
            pltpu.sync_copy(bv, hat.at[pl.ds(base, CHA)])

        def _row2(j, carry):
            for q in range(8):
                sl = pl.ds(16 * q, 16)
                bv[j, sl] = 1.0 / (1.0 + jnp.exp(-bv[j, sl]))
            return carry
        lax.fori_loop(0, CHA, _row2, 0)
        pltpu.sync_copy(bv, sig.at[pl.ds(base, CHA)])
        pltpu.sync_copy(bv, acc.at[idr], add=True)
        return bn

    bn0 = tuple(jnp.zeros((16,), _F32) for _ in range(16))
    _issue(ebase, isA, idA, r1A, r2A, s1A, s2A)

    def _pair(k, bn):
        i0 = ebase + (2 * k) * CHA
        i1 = i0 + CHA
        inext = jnp.minimum(i1 + CHA, ebase + (NFA - 2) * CHA)
        _issue(i1, isB, idB, r1B, r2B, s1B, s2B)
        bn = _process(i0, isA, idA, r1A, r2A, s1A, s2A, bn)
        _issue(inext, isA, idA, r1A, r2A, s1A, s2A)
        bn = _process(i1, isB, idB, r1B, r2B, s1B, s2B, bn)
        return bn

    bn = lax.fori_loop(0, NFA // 2, _pair, bn0)
    _wait(isA, idA, r1A, r2A, s1A, s2A)

    rbase = pl.multiple_of(ebase + NFA * CHA, 8)
    pltpu.sync_copy(src.at[pl.ds(rbase, REM)], is2)
    pltpu.sync_copy(dst.at[pl.ds(rbase, REM)], id2)
    pltpu.async_copy(t1.at[is2], r1A.at[pl.ds(0, REM)], s1A)
    pltpu.async_copy(t2.at[id2], r2A.at[pl.ds(0, REM)], s2A)
    pltpu.sync_copy(b3e.at[pl.ds(rbase, REM)], bv.at[pl.ds(0, REM)])
    pltpu.make_async_copy(t1.at[is2], r1A.at[pl.ds(0, REM)], s1A).wait()
    pltpu.make_async_copy(t2.at[id2], r2A.at[pl.ds(0, REM)], s2A).wait()

    def _rrow1(j, bnc):
        nb = list(bnc)
        for q in range(8):
            sl = pl.ds(16 * q, 16)
            a = r1A[j, sl] + r2A[j, sl] + bv[j, sl]
            bv[j, sl] = a
            nb[q] = nb[q] + a
            nb[8 + q] = nb[8 + q] + a * a
        return tuple(nb)
    bn = lax.fori_loop(0, REM, _rrow1, bn)
    if hat is not None:
        pltpu.sync_copy(bv.at[pl.ds(0, REM)], hat.at[pl.ds(rbase, REM)])

    def _rrow2(j, carry):
        for q in range(8):
            sl = pl.ds(16 * q, 16)
            bv[j, sl] = 1.0 / (1.0 + jnp.exp(-bv[j, sl]))
        return carry
    lax.fori_loop(0, REM, _rrow2, 0)
    pltpu.sync_copy(bv.at[pl.ds(0, REM)], sig.at[pl.ds(rbase, REM)])
    pltpu.sync_copy(bv.at[pl.ds(0, REM)], acc.at[id2], add=True)

    for q in range(8):
        bnbuf[0, pl.ds(16 * q, 16)] = bn[q]
        bnbuf[1, pl.ds(16 * q, 16)] = bn[8 + q]
    pltpu.sync_copy(bnbuf, bnpart.at[wid])

    plsc.subcore_barrier()
    _dump_acc(r1A, acc, sigpart, c, s)


def _sc_pass_b_body(src, dst, sig, tab,
                    accpart,
                    acc, isA, idA, isB, idB, is2, id2,
                    rtA, rtB, sgv, s1A, s1B):
    c = lax.axis_index("c")
    s = lax.axis_index("s")
    ebase = (s * NC + c) * EPT

    _zero_acc(sgv, acc, s)
    plsc.subcore_barrier()

    def _issue(base, isr, idr, gt, sm):
        base = pl.multiple_of(base, 8)
        pltpu.sync_copy(src.at[pl.ds(base, CHB)], isr)
        pltpu.sync_copy(dst.at[pl.ds(base, CHB)], idr)
        pltpu.async_copy(tab.at[isr], gt, sm)

    def _process(base, isr, idr, gt, sm):
        base = pl.multiple_of(base, 8)
        pltpu.sync_copy(sig.at[pl.ds(base, CHB)], sgv)
        pltpu.make_async_copy(tab.at[isr], gt, sm).wait()

        def _row(j, carry):
            for q in range(8):
                sl = pl.ds(16 * q, 16)
                sgv[j, sl] = sgv[j, sl] * gt[j, sl]
            return carry
        lax.fori_loop(0, CHB, _row, 0)
        pltpu.sync_copy(sgv, acc.at[idr], add=True)

    _issue(ebase, isA, idA, rtA, s1A)

    def _pair(k, carry):
        i0 = ebase + (2 * k) * CHB
        i1 = i0 + CHB
        inext = jnp.minimum(i1 + CHB, ebase + (NFB - 2) * CHB)
        _issue(i1, isB, idB, rtB, s1B)
        _process(i0, isA, idA, rtA, s1A)
        _issue(inext, isA, idA, rtA, s1A)
        _process(i1, isB, idB, rtB, s1B)
        return carry

    lax.fori_loop(0, NFB // 2, _pair, 0)
    pltpu.make_async_copy(tab.at[isA], rtA, s1A).wait()

    rbase = pl.multiple_of(ebase + NFB * CHB, 8)
    pltpu.sync_copy(src.at[pl.ds(rbase, REM)], is2)
    pltpu.sync_copy(dst.at[pl.ds(rbase, REM)], id2)
    pltpu.async_copy(tab.at[is2], rtA.at[pl.ds(0, REM)], s1A)
    pltpu.sync_copy(sig.at[pl.ds(rbase, REM)], sgv.at[pl.ds(0, REM)])
    pltpu.make_async_copy(tab.at[is2], rtA.at[pl.ds(0, REM)], s1A).wait()

    def _rrow(j, carry):
        for q in range(8):
            sl = pl.ds(16 * q, 16)
            sgv[j, sl] = sgv[j, sl] * rtA[j, sl]
        return carry
    lax.fori_loop(0, REM, _rrow, 0)
    pltpu.sync_copy(sgv.at[pl.ds(0, REM)], acc.at[id2], add=True)

    plsc.subcore_barrier()
    _dump_acc(sgv, acc, accpart, c, s)


_SC_MESH = plsc.VectorSubcoreMesh(
    core_axis_name="c", subcore_axis_name="s", num_cores=NC, num_subcores=NS)

_A_SCRATCH = [
    pltpu.VMEM_SHARED((N, D), _F32),
    pltpu.VMEM((CHA,), jnp.int32), pltpu.VMEM((CHA,), jnp.int32),
    pltpu.VMEM((CHA,), jnp.int32), pltpu.VMEM((CHA,), jnp.int32),
    pltpu.VMEM((REM,), jnp.int32), pltpu.VMEM((REM,), jnp.int32),
    pltpu.VMEM((CHA, D), _F32), pltpu.VMEM((CHA, D), _F32),
    pltpu.VMEM((CHA, D), _F32), pltpu.VMEM((CHA, D), _F32),
    pltpu.VMEM((CHA, D), _F32),
    pltpu.VMEM((2, D), _F32),
    pltpu.SemaphoreType.DMA, pltpu.SemaphoreType.DMA,
    pltpu.SemaphoreType.DMA, pltpu.SemaphoreType.DMA,
]

_sc_pass_a1 = pl.kernel(
    functools.partial(_sc_pass_a_body, True),
    out_type=(_sds((E, D)), _sds((E, D)), _sds((NC, N, D)), _sds((NW, 2, D))),
    mesh=_SC_MESH,
    scratch_types=list(_A_SCRATCH),
)

_sc_pass_a2 = pl.kernel(
    functools.partial(_sc_pass_a_body, False),
    out_type=(_sds((E, D)), _sds((NC, N, D)), _sds((NW, 2, D))),
    mesh=_SC_MESH,
    scratch_types=list(_A_SCRATCH),
)

_sc_pass_b = pl.kernel(
    _sc_pass_b_body,
    out_type=_sds((NC, N, D)),
    mesh=_SC_MESH,
    scratch_types=[
        pltpu.VMEM_SHARED((N, D), _F32),
        pltpu.VMEM((CHB,), jnp.int32), pltpu.VMEM((CHB,), jnp.int32),
        pltpu.VMEM((CHB,), jnp.int32), pltpu.VMEM((CHB,), jnp.int32),
        pltpu.VMEM((REM,), jnp.int32), pltpu.VMEM((REM,), jnp.int32),
        pltpu.VMEM((CHB, D), _F32), pltpu.VMEM((CHB, D), _F32),
        pltpu.VMEM((CHB, D), _F32),
        pltpu.SemaphoreType.DMA, pltpu.SemaphoreType.DMA,
    ],
)


def _dot(a, b):
    return jnp.dot(a, b, preferred_element_type=_F32)


def _tc_embed_body(nf, pe, ehw, ehb, epw, epb, h0, p0):
    h0[...] = _dot(nf[...], ehw[...]) + ehb[...]
    p0[...] = _dot(pe[...], epw[...]) + epb[...]


_tc_embed = pl.pallas_call(
    _tc_embed_body,
    grid=(NB,),
    in_specs=[
        pl.BlockSpec((NBR, NF), lambda i: (i, 0)),
        pl.BlockSpec((NBR, POS), lambda i: (i, 0)),
        pl.BlockSpec((NF, D), lambda i: (0, 0)),
        pl.BlockSpec((1, D), lambda i: (0, 0)),
        pl.BlockSpec((POS, D), lambda i: (0, 0)),
        pl.BlockSpec((1, D), lambda i: (0, 0)),
    ],
    out_specs=[pl.BlockSpec((NBR, D), lambda i: (i, 0))] * 2,
    out_shape=(_sds((N, D)), _sds((N, D))),
)


def _tc_tables_body(h, p, a1w, a1b, b1w, b1b, b2w, b2b, c1w, c1b,
                    a2w, a2b, c2w, c2b,
                    a1h, b1t, b2t, c1p, vt, c2p):
    hh = h[...]
    pp = p[...]
    hp = jnp.concatenate([hh, pp], axis=-1)
    a1h[...] = _dot(hp, a1w[...]) + a1b[...]
    b1t[...] = _dot(hh, b1w[...]) + b1b[...]
    b2t[...] = _dot(hh, b2w[...]) + b2b[...]
    c1p[...] = _dot(pp, c1w[...]) + c1b[...]
    vt[...] = _dot(hp, a2w[...]) + a2b[...]
    c2p[...] = _dot(pp, c2w[...]) + c2b[...]


_tc_tables = pl.pallas_call(
    _tc_tables_body,
    grid=(NB,),
    in_specs=[pl.BlockSpec((NBR, D), lambda i: (i, 0))] * 2 + [
        pl.BlockSpec((2 * D, D), lambda i: (0, 0)),
        pl.BlockSpec((1, D), lambda i: (0, 0)),
        pl.BlockSpec((D, D), lambda i: (0, 0)),
        pl.BlockSpec((1, D), lambda i: (0, 0)),
        pl.BlockSpec((D, D), lambda i: (0, 0)),
        pl.BlockSpec((1, D), lambda i: (0, 0)),
        pl.BlockSpec((D, D), lambda i: (0, 0)),
        pl.BlockSpec((1, D), lambda i: (0, 0)),
        pl.BlockSpec((2 * D, D), lambda i: (0, 0)),
        pl.BlockSpec((1, D), lambda i: (0, 0)),
        pl.BlockSpec((D, D), lambda i: (0, 0)),
        pl.BlockSpec((1, D), lambda i: (0, 0)),
    ],
    out_specs=[pl.BlockSpec((NBR, D), lambda i: (i, 0))] * 6,
    out_shape=tuple(_sds((N, D)) for _ in range(6)),
)


def _tc_b3e1_body(ef, ew, eb, b3w, b3b, out):
    u = _dot(ew[...], b3w[...])
    cst = _dot(eb[...], b3w[...]) + b3b[...]
    out[...] = ef[...] * u + cst


_tc_b3e1 = pl.pallas_call(
    _tc_b3e1_body,
    grid=(EG,),
    in_specs=[
        pl.BlockSpec((EBLK, 1), lambda i: (i, 0)),
        pl.BlockSpec((1, D), lambda i: (0, 0)),
        pl.BlockSpec((1, D), lambda i: (0, 0)),
        pl.BlockSpec((D, D), lambda i: (0, 0)),
        pl.BlockSpec((1, D), lambda i: (0, 0)),
    ],
    out_specs=pl.BlockSpec((EBLK, D), lambda i: (i, 0)),
    out_shape=_sds((E, D)),
)


def _tc_combine_body(sp, bp, ss, mv):
    spv = sp[...]
    ss[...] = spv[0] + spv[1]
    bpv = bp[...]
    m = jnp.sum(bpv[:, 0, :], axis=0) / float(E)
    q = jnp.sum(bpv[:, 1, :], axis=0) / float(E)
    mv[...] = jnp.stack([m, q - m * m], axis=0)


_tc_combine = pl.pallas_call(
    _tc_combine_body,
    out_shape=(_sds((N, D)), _sds((2, D))),
)


def _tc_b3e2_body(hat, ef, mv, ew, eb, ge, be, b3w, b3b, out):
    mvv = mv[...]
    m = mvv[0:1, :]
    v = mvv[1:2, :]
    xn = (hat[...] - m) / jnp.sqrt(v + 1e-5) * ge[...] + be[...]
    e2 = ef[...] * ew[...] + eb[...] + jnp.maximum(xn, 0.0)
    out[...] = _dot(e2, b3w[...]) + b3b[...]


_tc_b3e2 = pl.pallas_call(
    _tc_b3e2_body,
    grid=(EG,),
    in_specs=[
        pl.BlockSpec((EBLK, D), lambda i: (i, 0)),
        pl.BlockSpec((EBLK, 1), lambda i: (i, 0)),
        pl.BlockSpec((2, D), lambda i: (0, 0)),
        pl.BlockSpec((1, D), lambda i: (0, 0)),
        pl.BlockSpec((1, D), lambda i: (0, 0)),
        pl.BlockSpec((1, D), lambda i: (0, 0)),
        pl.BlockSpec((1, D), lambda i: (0, 0)),
        pl.BlockSpec((D, D), lambda i: (0, 0)),
        pl.BlockSpec((1, D), lambda i: (0, 0)),
    ],
    out_specs=pl.BlockSpec((EBLK, D), lambda i: (i, 0)),
    out_shape=_sds((E, D)),
)


def _tc_hupd_body(a1h, hacc, ss, sn, hin, g, b, out):
    ha = hacc[...]
    t = (a1h[...] + (ha[0] + ha[1]) / (ss[...] + 1e-6)) * sn[...]
    m = jnp.mean(t, axis=0, keepdims=True)
    v = jnp.mean(t * t, axis=0, keepdims=True) - m * m
    out[...] = hin[...] + jnp.maximum(
        (t - m) / jnp.sqrt(v + 1e-5) * g[...] + b[...], 0.0)


_tc_hupd = pl.pallas_call(_tc_hupd_body, out_shape=_sds((N, D)))


def _tc_pupd_body(c1p, pacc, ss, pin, out):
    pa = pacc[...]
    out[...] = pin[...] + jnp.tanh(
        c1p[...] + (pa[0] + pa[1]) / (ss[...] + 1e-6))


_tc_pupd = pl.pallas_call(_tc_pupd_body, out_shape=_sds((N, D)))


def _tc_head_body(a1h2, hacc, ss2, sn, h1, g, b, p3r,
                  pow_, pob, whpw, whpb, w1, b1_, w2, b2_, w3, b3_, out):
    ha = hacc[...]
    ssv = ss2[...] + 1e-6
    t = (a1h2[...] + (ha[0] + ha[1]) / ssv) * sn[...]
    m = jnp.mean(t, axis=0, keepdims=True)
    v = jnp.mean(t * t, axis=0, keepdims=True) - m * m
    h3 = h1[...] + jnp.maximum(
        (t - m) / jnp.sqrt(v + 1e-5) * g[...] + b[...], 0.0)
    p3 = p3r[...]
    pp = _dot(p3, pow_[...]) + pob[...]
    pp = pp - jnp.mean(pp, axis=0, keepdims=True)
    pp = pp / jnp.sqrt(jnp.sum(pp * pp, axis=0, keepdims=True))
    hp0 = _dot(jnp.concatenate([h3[0:1], pp[0:1]], axis=-1),
               whpw[...]) + whpb[...]
    y = jnp.maximum(_dot(hp0, w1[...]) + b1_[...], 0.0)
    y = jnp.maximum(_dot(y, w2[...]) + b2_[...], 0.0)
    y = _dot(y, w3[...]) + b3_[...]
    out[...] = jnp.broadcast_to(y, (8, 128))


_tc_head = pl.pallas_call(_tc_head_body, out_shape=_sds((8, 128)))


def kernel(node_feat, pos_enc, edge_feat, snorm_n, targets, edge_index, params):
    lp1, lp2 = params['layers'][0], params['layers'][1]
    src = edge_index[0]
    dst = edge_index[1]

    def r2(x):
        return x.reshape(1, -1)

    h0, p0 = _tc_embed(node_feat, pos_enc,
                       params['emb_h_W'], r2(params['emb_h_b']),
                       params['emb_p_W'], r2(params['emb_p_b']))

    def tables(h, p, lp):
        return _tc_tables(h, p,
                          lp['A1_W'], r2(lp['A1_b']),
                          lp['B1_W'], r2(lp['B1_b']),
                          lp['B2_W'], r2(lp['B2_b']),
                          lp['C1_W'], r2(lp['C1_b']),
                          lp['A2_W'], r2(lp['A2_b']),
                          lp['C2_W'], r2(lp['C2_b']))

    a1h1, b11, b21, c1p1, v1, c2p1 = tables(h0, p0, lp1)
    b3e1 = _tc_b3e1(edge_feat, params['emb_e_W'], r2(params['emb_e_b']),
                    lp1['B3_W'], r2(lp1['B3_b']))

    hat1, sig1, sigp1, bnp1 = _sc_pass_a1(src, dst, b3e1, b11, b21)
    ss1, mv1 = _tc_combine(sigp1, bnp1)
    hacc1 = _sc_pass_b(src, dst, sig1, v1)
    pacc1 = _sc_pass_b(src, dst, sig1, c2p1)

    h1 = _tc_hupd(a1h1, hacc1, ss1, snorm_n, h0,
                  r2(lp1['bn_h_g']), r2(lp1['bn_h_b']))
    p1 = _tc_pupd(c1p1, pacc1, ss1, p0)
    b3e2 = _tc_b3e2(hat1, edge_feat, mv1,
                    params['emb_e_W'], r2(params['emb_e_b']),
                    r2(lp1['bn_e_g']), r2(lp1['bn_e_b']),
                    lp2['B3_W'], r2(lp2['B3_b']))

    a1h2, b12, b22, c1p2, v2, c2p2 = tables(h1, p1, lp2)
    sig2, sigp2, bnp2 = _sc_pass_a2(src, dst, b3e2, b12, b22)
    ss2, _unused = _tc_combine(sigp2, bnp2)
    hacc2 = _sc_pass_b(src, dst, sig2, v2)
    pacc2 = _sc_pass_b(src, dst, sig2, c2p2)

    p3 = _tc_pupd(c1p2, pacc2, ss2, p1)
    out = _tc_head(a1h2, hacc2, ss2, snorm_n, h1,
                   r2(lp2['bn_h_g']), r2(lp2['bn_h_b']),
                   p3,
                   params['p_out_W'], r2(params['p_out_b']),
                   params['Whp_W'], r2(params['Whp_b']),
                   params['mlp'][0][0], r2(params['mlp'][0][1]),
                   params['mlp'][1][0], r2(params['mlp'][1][1]),
                   params['mlp'][2][0], r2(params['mlp'][2][1]))
    scores = out[0:1, 0:1]
    return (scores, targets)
